# baseline (speedup 1.0000x reference)
import jax
import jax.numpy as jnp
from jax.experimental import pallas as pl
from jax.experimental.pallas import tpu as pltpu

_BLOCK_ROWS = 10000


def _body(x_ref, w_ref, b_ref, o_ref):
    xb = x_ref[...].astype(jnp.float32)
    acc = jax.lax.dot_general(
        xb, w_ref[...],
        dimension_numbers=(((1,), (0,)), ((), ())),
        preferred_element_type=jnp.float32,
    )
    o_ref[...] = acc + b_ref[...]


def kernel(x, tables):
    n, f = x.shape
    d = tables.shape[-1]
    t0 = tables[:, 0, :]
    diff = tables[:, 1, :] - t0
    base = jnp.sum(t0, axis=0, keepdims=True)
    z = jnp.zeros((f, d), jnp.float32)
    w2 = jnp.concatenate(
        [jnp.concatenate([diff, z], axis=1),
         jnp.concatenate([z, diff], axis=1)], axis=0)
    b2 = jnp.concatenate([base, base], axis=1)
    n2 = n // 2
    x2 = x.reshape(n2, 2 * f)
    grid = (n2 + _BLOCK_ROWS - 1) // _BLOCK_ROWS
    out2 = pl.pallas_call(
        _body,
        grid=(grid,),
        in_specs=[
            pl.BlockSpec((_BLOCK_ROWS, 2 * f), lambda i: (i, 0)),
            pl.BlockSpec((2 * f, 2 * d), lambda i: (0, 0)),
            pl.BlockSpec((1, 2 * d), lambda i: (0, 0)),
        ],
        out_specs=pl.BlockSpec((_BLOCK_ROWS, 2 * d), lambda i: (i, 0)),
        out_shape=jax.ShapeDtypeStruct((n2, 2 * d), jnp.float32),
    )(x2, w2, b2)
    return out2.reshape(n, d)


# speedup vs baseline: 2.3711x; 2.3711x over previous
import jax
import jax.numpy as jnp
from jax.experimental import pallas as pl
from jax.experimental.pallas import tpu as pltpu

_BLOCK_ROWS = 10000


def _body(x_ref, t_ref, o_ref):
    t0 = t_ref[0]
    t1 = t_ref[1]
    diff = (t1 - t0).astype(jnp.bfloat16)
    base = jnp.sum(t0, axis=0, keepdims=True)
    xb = x_ref[...].astype(jnp.bfloat16)
    acc = jax.lax.dot_general(
        xb, diff,
        dimension_numbers=(((1,), (0,)), ((), ())),
        preferred_element_type=jnp.float32,
    )
    o_ref[...] = acc + base


def kernel(x, tables):
    n, f = x.shape
    d = tables.shape[-1]
    tt = tables.transpose(1, 0, 2)
    grid = (n + _BLOCK_ROWS - 1) // _BLOCK_ROWS
    return pl.pallas_call(
        _body,
        grid=(grid,),
        in_specs=[
            pl.BlockSpec((_BLOCK_ROWS, f), lambda i: (i, 0)),
            pl.BlockSpec((2, f, d), lambda i: (0, 0, 0)),
        ],
        out_specs=pl.BlockSpec((_BLOCK_ROWS, d), lambda i: (i, 0)),
        out_shape=jax.ShapeDtypeStruct((n, d), jnp.float32),
    )(x, tt)
